# baseline (speedup 1.0000x reference)
import jax
import jax.numpy as jnp
from jax import lax
from jax.experimental import pallas as pl
from jax.experimental.pallas import tpu as pltpu
from jax.experimental.pallas import tpu_sc as plsc

NUM_EMBEDDINGS = 1000000
EMBEDDING_DIM = 32
BATCH = 16384
HIST = 50

NC, NS = 2, 16
NW = NC * NS
TOTAL = BATCH * HIST
PER_W = TOTAL // NW
C = 640
N_CHUNKS = PER_W // C


NB = 4


def _gather_body(table_hbm, xf_hbm, out_hbm,
                 idx_all, rows0, rows1, rows2, rows3,
                 sem_i, sem_g0, sem_g1, sem_g2, sem_g3,
                 sem_s0, sem_s1, sem_s2, sem_s3):
    wid = lax.axis_index("s") * NC + lax.axis_index("c")
    f0 = wid * PER_W
    rows = (rows0, rows1, rows2, rows3)
    sem_g = (sem_g0, sem_g1, sem_g2, sem_g3)
    sem_s = (sem_s0, sem_s1, sem_s2, sem_s3)

    pltpu.async_copy(xf_hbm.at[pl.ds(f0, PER_W)], idx_all, sem_i).wait()

    def gather(k, buf):
        return pltpu.async_copy(
            table_hbm.at[idx_all.at[pl.ds(k * C, C)]], rows[buf], sem_g[buf])

    def store_desc(k, buf):
        return pltpu.make_async_copy(
            rows[buf], out_hbm.at[pl.ds(f0 + k * C, C)], sem_s[buf])

    def group_body(g, carry):
        k0 = g * NB
        gs = []
        for j in range(NB):
            @pl.when(g > 0)
            def _(j=j, k0=k0):
                store_desc(k0 - NB + j, j).wait()
            gs.append(gather(k0 + j, j))
        for j in range(NB):
            gs[j].wait()
            store_desc(k0 + j, j).start()
        return carry

    lax.fori_loop(0, N_CHUNKS // NB, group_body, 0, unroll=False)
    for j in range(NB):
        store_desc(N_CHUNKS - NB + j, j).wait()


_sc_gather = pl.kernel(
    _gather_body,
    out_type=jax.ShapeDtypeStruct((TOTAL, EMBEDDING_DIM), jnp.float32),
    mesh=plsc.VectorSubcoreMesh(core_axis_name="c", subcore_axis_name="s"),
    scratch_types=(
        [pltpu.VMEM((PER_W,), jnp.int32)]
        + [pltpu.VMEM((C, EMBEDDING_DIM), jnp.float32) for _ in range(4)]
        + [pltpu.SemaphoreType.DMA for _ in range(9)]
    ),
    compiler_params=pltpu.CompilerParams(
        use_tc_tiling_on_sc=False, needs_layout_passes=False),
)


@jax.jit
def kernel(x, table):
    xf = x.reshape(TOTAL).astype(jnp.int32)
    out = _sc_gather(table, xf)
    return out.reshape(BATCH, HIST, EMBEDDING_DIM)


# speedup vs baseline: 1.3006x; 1.3006x over previous
import jax
import jax.numpy as jnp
from jax import lax
from jax.experimental import pallas as pl
from jax.experimental.pallas import tpu as pltpu
from jax.experimental.pallas import tpu_sc as plsc

NUM_EMBEDDINGS = 1000000
EMBEDDING_DIM = 32
BATCH = 16384
HIST = 50

NC, NS, L = 2, 16, 16
NW = NC * NS
BW = BATCH // NW
N_CHUNKS = HIST


def _transpose_chunk(rows, tbuf):
    def jb_body(jb, carry):
        row_idx = jb * L + lax.iota(jnp.int32, L)
        for d in range(EMBEDDING_DIM):
            col_idx = jnp.full((L,), d, jnp.int32)
            tbuf[d, pl.ds(jb * L, L)] = plsc.load_gather(rows, [row_idx, col_idx])
        return carry

    lax.fori_loop(0, BW // L, jb_body, 0, unroll=False)


def _gather_body(table_hbm, xt_hbm, out_hbm,
                 idx_all, rows_v0, rows_v1, tbuf0, tbuf1,
                 sem_i, sem_g0, sem_g1, sem_s0, sem_s1):
    wid = lax.axis_index("s") * NC + lax.axis_index("c")
    b0 = wid * BW
    rows_v = (rows_v0, rows_v1)
    tbuf = (tbuf0, tbuf1)
    sem_g = (sem_g0, sem_g1)
    sem_s = (sem_s0, sem_s1)

    pltpu.async_copy(xt_hbm.at[:, pl.ds(b0, BW)], idx_all, sem_i).wait()

    def gather(h, buf):
        return pltpu.async_copy(table_hbm.at[idx_all.at[h]], rows_v[buf], sem_g[buf])

    def store_desc(h, buf):
        return pltpu.make_async_copy(
            tbuf[buf], out_hbm.at[h, :, pl.ds(b0, BW)], sem_s[buf])

    def pair_body(k, carry):
        ha = 2 * k
        hb = 2 * k + 1
        ga = gather(ha, 0)
        gb = gather(hb, 1)

        @pl.when(k > 0)
        def _():
            store_desc(2 * k - 2, 0).wait()
            store_desc(2 * k - 1, 1).wait()

        ga.wait()
        _transpose_chunk(rows_v[0], tbuf[0])
        store_desc(ha, 0).start()
        gb.wait()
        _transpose_chunk(rows_v[1], tbuf[1])
        store_desc(hb, 1).start()
        return carry

    lax.fori_loop(0, N_CHUNKS // 2, pair_body, 0, unroll=False)
    store_desc(N_CHUNKS - 2, 0).wait()
    store_desc(N_CHUNKS - 1, 1).wait()


_sc_gather = pl.kernel(
    _gather_body,
    out_type=jax.ShapeDtypeStruct((HIST, EMBEDDING_DIM, BATCH), jnp.float32),
    mesh=plsc.VectorSubcoreMesh(core_axis_name="c", subcore_axis_name="s"),
    scratch_types=[
        pltpu.VMEM((HIST, BW), jnp.int32),
        pltpu.VMEM((BW, EMBEDDING_DIM), jnp.float32),
        pltpu.VMEM((BW, EMBEDDING_DIM), jnp.float32),
        pltpu.VMEM((EMBEDDING_DIM, BW), jnp.float32),
        pltpu.VMEM((EMBEDDING_DIM, BW), jnp.float32),
        pltpu.SemaphoreType.DMA,
        pltpu.SemaphoreType.DMA,
        pltpu.SemaphoreType.DMA,
        pltpu.SemaphoreType.DMA,
        pltpu.SemaphoreType.DMA,
    ],
    compiler_params=pltpu.CompilerParams(
        use_tc_tiling_on_sc=False, needs_layout_passes=False),
)


@jax.jit
def kernel(x, table):
    xt = x.T.astype(jnp.int32)
    out_t = _sc_gather(table, xt)
    return jnp.transpose(out_t, (2, 0, 1))


# speedup vs baseline: 1.3500x; 1.0380x over previous
import jax
import jax.numpy as jnp
from jax import lax
from jax.experimental import pallas as pl
from jax.experimental.pallas import tpu as pltpu
from jax.experimental.pallas import tpu_sc as plsc

NUM_EMBEDDINGS = 1000000
EMBEDDING_DIM = 32
BATCH = 16384
HIST = 50

NC, NS, L = 2, 16, 16
NW = NC * NS
BW = BATCH // NW
N_CHUNKS = HIST
NB = 4


def _transpose_chunk(rows, tbuf):
    def jb_body(jb, carry):
        row_idx = jb * L + lax.iota(jnp.int32, L)
        for d in range(EMBEDDING_DIM):
            col_idx = jnp.full((L,), d, jnp.int32)
            tbuf[d, pl.ds(jb * L, L)] = plsc.load_gather(rows, [row_idx, col_idx])
        return carry

    lax.fori_loop(0, BW // L, jb_body, 0, unroll=False)


def _gather_body(table_hbm, xt_hbm, out_hbm,
                 idx_all, rv0, rv1, rv2, rv3, tb0, tb1,
                 sem_i, sg0, sg1, sg2, sg3, ss0, ss1):
    wid = lax.axis_index("s") * NC + lax.axis_index("c")
    b0 = wid * BW
    rv = (rv0, rv1, rv2, rv3)
    sg = (sg0, sg1, sg2, sg3)
    tb = (tb0, tb1)
    ss = (ss0, ss1)

    pltpu.async_copy(xt_hbm.at[:, pl.ds(b0, BW)], idx_all, sem_i).wait()

    def gather_desc(h, j):
        return pltpu.make_async_copy(table_hbm.at[idx_all.at[h]], rv[j], sg[j])

    def store_desc(h, t):
        return pltpu.make_async_copy(
            tb[t], out_hbm.at[h, :, pl.ds(b0, BW)], ss[t])

    for j in range(NB):
        gather_desc(j, j).start()

    def group_body(g, carry):
        k0 = NB * g
        for j in range(NB):
            k = k0 + j

            @pl.when(k < N_CHUNKS)
            def _(k=k, j=j):
                gather_desc(k, j).wait()

                @pl.when(k >= 2)
                def _(k=k, j=j):
                    store_desc(k - 2, j % 2).wait()

                _transpose_chunk(rv[j], tb[j % 2])
                store_desc(k, j % 2).start()

                @pl.when(k + NB < N_CHUNKS)
                def _(k=k, j=j):
                    gather_desc(k + NB, j).start()

        return carry

    n_groups = (N_CHUNKS + NB - 1) // NB
    lax.fori_loop(0, n_groups, group_body, 0, unroll=False)
    store_desc(N_CHUNKS - 2, 0).wait()
    store_desc(N_CHUNKS - 1, 1).wait()


_sc_gather = pl.kernel(
    _gather_body,
    out_type=jax.ShapeDtypeStruct((HIST, EMBEDDING_DIM, BATCH), jnp.float32),
    mesh=plsc.VectorSubcoreMesh(core_axis_name="c", subcore_axis_name="s"),
    scratch_types=(
        [pltpu.VMEM((HIST, BW), jnp.int32)]
        + [pltpu.VMEM((BW, EMBEDDING_DIM), jnp.float32) for _ in range(NB)]
        + [pltpu.VMEM((EMBEDDING_DIM, BW), jnp.float32) for _ in range(2)]
        + [pltpu.SemaphoreType.DMA for _ in range(1 + NB + 2)]
    ),
    compiler_params=pltpu.CompilerParams(
        use_tc_tiling_on_sc=False, needs_layout_passes=False),
)


@jax.jit
def kernel(x, table):
    xt = x.T.astype(jnp.int32)
    out_t = _sc_gather(table, xt)
    return jnp.transpose(out_t, (2, 0, 1))
